# baseline (speedup 1.0000x reference)
import functools

import jax
import jax.numpy as jnp
from jax import lax
from jax.experimental import pallas as pl
from jax.experimental.pallas import tpu as pltpu
from jax.experimental.pallas import tpu_sc as plsc

VOCAB = 1000000
EMB = 64
BATCH = 16384
NEG = 20
NP1 = NEG + 1

NC, NS = 2, 16
NW = NC * NS

BW = BATCH // NW
GB = 32
NBLK = BW // GB
CH = 128
TROWS = BW // CH
NRB = GB * NEG // CH
NROWS = NRB * NBLK

DCOLS = 128
DROWS = BATCH * NP1 // DCOLS
DW = DROWS // NW


def _dots_body(emb_in, emb_out, idx_t, idx_c, idx_n, dots_out,
               itv, icv, inv, vbuf, ubuf, nbuf, dots, sem0, sem1):
    c = lax.axis_index("c")
    s = lax.axis_index("s")
    wid = s * NC + c

    pltpu.sync_copy(idx_t.at[pl.ds(wid * TROWS, TROWS)], itv)
    pltpu.sync_copy(idx_c.at[pl.ds(wid * TROWS, TROWS)], icv)
    pltpu.sync_copy(idx_n.at[pl.ds(wid * NROWS, NROWS)], inv)

    def copies(blk, buf):
        r = blk >> 2
        co = (blk & 3) * GB
        yield emb_in.at[itv.at[r, pl.ds(co, GB)]], vbuf.at[buf]
        yield emb_out.at[icv.at[r, pl.ds(co, GB)]], ubuf.at[buf]
        for i in range(NRB):
            yield (emb_out.at[inv.at[blk * NRB + i]],
                   nbuf.at[buf, pl.ds(i * CH, CH)])

    def start(blk, buf, sem):
        for src, dst in copies(blk, buf):
            pltpu.async_copy(src, dst, sem)

    def drain(blk, buf, sem):
        for src, dst in copies(blk, buf):
            pltpu.make_async_copy(src, dst, sem).wait()

    last = lax.iota(jnp.int32, 16) == 15
    lane = lax.iota(jnp.int32, 16)
    perms = [(lane + h) % 16 for h in (8, 4, 2, 1)]

    def compute(blk, buf):
        vb, ub, nb = vbuf.at[buf], ubuf.at[buf], nbuf.at[buf]

        def bstep(b, _):
            vr = [vb[b, pl.ds(c * 16, 16)] for c in range(EMB // 16)]
            flat = (blk * GB + b) * NP1

            def emit(ref, row, f):
                s = ref[row, pl.ds(0, 16)] * vr[0]
                for c in range(1, EMB // 16):
                    s = s + ref[row, pl.ds(c * 16, 16)] * vr[c]
                for p in perms:
                    s = s + jnp.take_along_axis(s, p, axis=0)
                fv = jnp.full((16,), f, jnp.int32)
                plsc.store_scatter(dots, [fv >> 7, fv & 127], s, mask=last)

            emit(ub, b, flat)
            for k in range(NEG):
                emit(nb, b * NEG + k, flat + 1 + k)
            return 0

        lax.fori_loop(0, GB, bstep, 0)

    start(0, 0, sem0)

    def pairstep(p, _):
        blk = 2 * p
        start(blk + 1, 1, sem1)
        drain(blk, 0, sem0)
        compute(blk, 0)

        @pl.when(blk + 2 < NBLK)
        def _():
            start(blk + 2, 0, sem0)

        drain(blk + 1, 1, sem1)
        compute(blk + 1, 1)
        return 0

    lax.fori_loop(0, NBLK // 2, pairstep, 0)

    pltpu.sync_copy(dots, dots_out.at[pl.ds(wid * DW, DW)])


_dots = functools.partial(
    pl.kernel,
    out_type=jax.ShapeDtypeStruct((DROWS, DCOLS), jnp.float32),
    mesh=plsc.VectorSubcoreMesh(core_axis_name="c", subcore_axis_name="s",
                                num_cores=NC, num_subcores=NS),
    scratch_types=[
        pltpu.VMEM((TROWS, CH), jnp.int32),
        pltpu.VMEM((TROWS, CH), jnp.int32),
        pltpu.VMEM((NROWS, CH), jnp.int32),
        pltpu.VMEM((2, GB, EMB), jnp.float32),
        pltpu.VMEM((2, GB, EMB), jnp.float32),
        pltpu.VMEM((2, GB * NEG, EMB), jnp.float32),
        pltpu.VMEM((DW, DCOLS), jnp.float32),
        pltpu.SemaphoreType.DMA,
        pltpu.SemaphoreType.DMA,
    ],
    compiler_params=pltpu.CompilerParams(use_tc_tiling_on_sc=False,
                                         needs_layout_passes=False),
)(_dots_body)


def _loss_body(d_ref, out_ref):
    x = d_ref[...]
    r = lax.broadcasted_iota(jnp.int32, (DROWS, DCOLS), 0)
    col = lax.broadcasted_iota(jnp.int32, (DROWS, DCOLS), 1)
    flat = r * DCOLS + col
    y = jnp.where(flat % NP1 == 0, x, -x)
    ls = jnp.minimum(y, 0.0) - jnp.log1p(jnp.exp(-jnp.abs(y)))
    out_ref[0, 0] = -jnp.sum(ls) / BATCH


_loss = pl.pallas_call(
    _loss_body,
    in_specs=[pl.BlockSpec((DROWS, DCOLS), lambda: (0, 0))],
    out_specs=pl.BlockSpec(memory_space=pltpu.SMEM),
    out_shape=jax.ShapeDtypeStruct((1, 1), jnp.float32),
)


def kernel(target_input, context, neg, input_emb, output_emb):
    idx_t = (target_input.astype(jnp.int32) * 2).reshape(BATCH // CH, CH)
    idx_c = (context.astype(jnp.int32) * 2).reshape(BATCH // CH, CH)
    idx_n = (neg.astype(jnp.int32) * 2).reshape(BATCH * NEG // CH, CH)
    dots = _dots(input_emb, output_emb, idx_t, idx_c, idx_n)
    return _loss(dots)[0, 0]


# speedup vs baseline: 1.0266x; 1.0266x over previous
import functools

import jax
import jax.numpy as jnp
from jax import lax
from jax.experimental import pallas as pl
from jax.experimental.pallas import tpu as pltpu
from jax.experimental.pallas import tpu_sc as plsc

VOCAB = 1000000
EMB = 64
BATCH = 16384
NEG = 20
NP1 = NEG + 1

NC, NS = 2, 16
NW = NC * NS

BW = BATCH // NW
GB = 32
NBLK = BW // GB
CH = 128
TROWS = BW // CH
NRB = GB * NEG // CH
NROWS = NRB * NBLK

DCOLS = 128
DROWS = BATCH * NP1 // DCOLS
DW = DROWS // NW


def _dots_body(emb_in, emb_out, idx_t, idx_c, idx_n, dots_out,
               itv, icv, inv, vbuf, ubuf, nbuf, dots, sem0, sem1):
    c = lax.axis_index("c")
    s = lax.axis_index("s")
    wid = s * NC + c

    pltpu.sync_copy(idx_t.at[pl.ds(wid * TROWS, TROWS)], itv)
    pltpu.sync_copy(idx_c.at[pl.ds(wid * TROWS, TROWS)], icv)
    pltpu.sync_copy(idx_n.at[:, pl.ds(wid * BW, BW)], inv)

    def copies(blk, buf):
        r = blk >> 2
        co = (blk & 3) * GB
        yield emb_in.at[itv.at[r, pl.ds(co, GB)]], vbuf.at[buf]
        yield emb_out.at[icv.at[r, pl.ds(co, GB)]], ubuf.at[buf]
        for k in range(NEG):
            yield (emb_out.at[inv.at[k, pl.ds(blk * GB, GB)]],
                   nbuf.at[buf, k])

    def start(blk, buf, sem):
        for src, dst in copies(blk, buf):
            pltpu.async_copy(src, dst, sem)

    def drain(blk, buf, sem):
        for src, dst in copies(blk, buf):
            pltpu.make_async_copy(src, dst, sem).wait()

    last = lax.iota(jnp.int32, 16) == 15

    def compute(blk, buf):
        vb, ub, nb = vbuf.at[buf], ubuf.at[buf], nbuf.at[buf]

        def bstep(b, _):
            vr = [vb[b, pl.ds(c * 16, 16)] for c in range(EMB // 16)]
            flat = (blk * GB + b) * NP1

            def emit(ref_row, f):
                s = ref_row[pl.ds(0, 16)] * vr[0]
                for c in range(1, EMB // 16):
                    s = s + ref_row[pl.ds(c * 16, 16)] * vr[c]
                cs = plsc.cumsum(s)
                fv = jnp.full((16,), f, jnp.int32)
                plsc.store_scatter(dots, [fv >> 7, fv & 127], cs, mask=last)

            emit(ub.at[b], flat)
            for k in range(NEG):
                emit(nb.at[k, b], flat + 1 + k)
            return 0

        lax.fori_loop(0, GB, bstep, 0)

    start(0, 0, sem0)

    def pairstep(p, _):
        blk = 2 * p
        start(blk + 1, 1, sem1)
        drain(blk, 0, sem0)
        compute(blk, 0)

        @pl.when(blk + 2 < NBLK)
        def _():
            start(blk + 2, 0, sem0)

        drain(blk + 1, 1, sem1)
        compute(blk + 1, 1)
        return 0

    lax.fori_loop(0, NBLK // 2, pairstep, 0)

    pltpu.sync_copy(dots, dots_out.at[pl.ds(wid * DW, DW)])


_dots = functools.partial(
    pl.kernel,
    out_type=jax.ShapeDtypeStruct((DROWS, DCOLS), jnp.float32),
    mesh=plsc.VectorSubcoreMesh(core_axis_name="c", subcore_axis_name="s",
                                num_cores=NC, num_subcores=NS),
    scratch_types=[
        pltpu.VMEM((TROWS, CH), jnp.int32),
        pltpu.VMEM((TROWS, CH), jnp.int32),
        pltpu.VMEM((NEG, BW), jnp.int32),
        pltpu.VMEM((2, GB, EMB), jnp.float32),
        pltpu.VMEM((2, GB, EMB), jnp.float32),
        pltpu.VMEM((2, NEG, GB, EMB), jnp.float32),
        pltpu.VMEM((DW, DCOLS), jnp.float32),
        pltpu.SemaphoreType.DMA,
        pltpu.SemaphoreType.DMA,
    ],
    compiler_params=pltpu.CompilerParams(use_tc_tiling_on_sc=False,
                                         needs_layout_passes=False),
)(_dots_body)


def _loss_body(d_ref, out_ref):
    x = d_ref[...]
    r = lax.broadcasted_iota(jnp.int32, (DROWS, DCOLS), 0)
    col = lax.broadcasted_iota(jnp.int32, (DROWS, DCOLS), 1)
    flat = r * DCOLS + col
    y = jnp.where(flat % NP1 == 0, x, -x)
    ls = jnp.minimum(y, 0.0) - jnp.log1p(jnp.exp(-jnp.abs(y)))
    out_ref[0, 0] = -jnp.sum(ls) / BATCH


_loss = pl.pallas_call(
    _loss_body,
    in_specs=[pl.BlockSpec((DROWS, DCOLS), lambda: (0, 0))],
    out_specs=pl.BlockSpec(memory_space=pltpu.SMEM),
    out_shape=jax.ShapeDtypeStruct((1, 1), jnp.float32),
)


def kernel(target_input, context, neg, input_emb, output_emb):
    idx_t = (target_input.astype(jnp.int32) * 2).reshape(BATCH // CH, CH)
    idx_c = (context.astype(jnp.int32) * 2).reshape(BATCH // CH, CH)
    idx_n = neg.astype(jnp.int32).T * 2
    dots = _dots(input_emb, output_emb, idx_t, idx_c, idx_n)
    return _loss(dots)[0, 0]
